# baseline (speedup 1.0000x reference)
import functools
import math

import jax
import jax.numpy as jnp
import numpy as np
from jax import lax
from jax.experimental import pallas as pl
from jax.experimental.pallas import tpu as pltpu
from jax.experimental.pallas import tpu_sc as plsc

N = 65536
NCHUNK = 32
CH = 2048
LANES = 16


def _rev_bits(x, nbits):
    r = np.zeros_like(x)
    t = x.copy()
    for _ in range(nbits):
        r = (r << 1) | (t & 1)
        t >>= 1
    return r

_BITREV_IDX = _rev_bits(np.arange(N, dtype=np.int64), 16).reshape(
    NCHUNK, LANES, 128).astype(np.int32)

_lane = np.arange(LANES, dtype=np.int64)
_wr163, _wi163 = [], []
for _s in range(1, 4):
    _h = 1 << _s
    _ang = -2.0 * np.pi * (_lane & (_h - 1)) / (2 * _h)
    _wr163.append(np.cos(_ang))
    _wi163.append(np.sin(_ang))
_LOC_OFF = {}
_twc, _tws = [], []
_o = 0
for _s in range(4, 11):
    _h = 1 << _s
    _ang = -2.0 * np.pi * np.arange(_h, dtype=np.float64) / (2 * _h)
    _twc.append(np.cos(_ang))
    _tws.append(np.sin(_ang))
    _LOC_OFF[_s] = _o
    _o += _h
_NLOC = _o
_WR163_OFF = 0
_WI163_OFF = 48
_TWC_OFF = 96
_TWS_OFF = 96 + _NLOC
_NCONST = 96 + 2 * _NLOC
_CONSTS = np.concatenate(
    _wr163 + _wi163 + _twc + _tws).astype(np.float32)
assert _CONSTS.shape == (_NCONST,)

_CROSS = {}
for _s in range(11, 16):
    _h = 1 << _s
    _ang = -2.0 * np.pi * np.arange(_h, dtype=np.float64) / (2 * _h)
    _CROSS[_s] = (np.cos(_ang).astype(np.float32),
                  np.sin(_ang).astype(np.float32))

_MESH = plsc.VectorSubcoreMesh(
    core_axis_name="c", subcore_axis_name="s", num_cores=2, num_subcores=16)


def _k1_body(re_hbm, im_hbm, idx_hbm, consts_hbm, ore_hbm, oim_hbm,
             idx_v, re_v, im_v, tw_v, sem):
    wid = lax.axis_index("c") * 16 + lax.axis_index("s")

    pltpu.sync_copy(idx_hbm.at[wid], idx_v)
    pltpu.sync_copy(consts_hbm, tw_v)

    copies = []
    for j in range(LANES):
        copies.append(pltpu.make_async_copy(
            re_hbm.at[idx_v.at[j]], re_v.at[pl.ds(j * 128, 128)], sem))
        copies.append(pltpu.make_async_copy(
            im_hbm.at[idx_v.at[j]], im_v.at[pl.ds(j * 128, 128)], sem))
    for c in copies:
        c.start()
    for c in copies:
        c.wait()

    iota = lax.iota(jnp.int32, LANES)

    for s in range(0, 4):
        h = 1 << s
        pat = ((iota >> s) << (s + 1)) + (iota & (h - 1))
        if s > 0:
            wr = tw_v[pl.ds(_WR163_OFF + (s - 1) * 16, 16)]
            wi = tw_v[pl.ds(_WI163_OFF + (s - 1) * 16, 16)]

        def body03(k, _, s=s, h=h, pat=pat,
                   wr=(None if s == 0 else wr), wi=(None if s == 0 else wi)):
            ti = k * 32 + pat
            bi_ = ti + h
            tr = plsc.load_gather(re_v, [ti])
            tii = plsc.load_gather(im_v, [ti])
            br = plsc.load_gather(re_v, [bi_])
            bii = plsc.load_gather(im_v, [bi_])
            if s == 0:
                xr, xi = br, bii
            else:
                xr = wr * br - wi * bii
                xi = wi * br + wr * bii
            plsc.store_scatter(re_v, [ti], tr + xr)
            plsc.store_scatter(im_v, [ti], tii + xi)
            plsc.store_scatter(re_v, [bi_], tr - xr)
            plsc.store_scatter(im_v, [bi_], tii - xi)
            return 0

        lax.fori_loop(0, 64, body03, 0)

    for s in range(4, 11):
        h = 1 << s

        def body(k, _, s=s, h=h):
            b = k * 16
            r = b & (h - 1)
            t0 = ((b >> s) << (s + 1)) + r
            b0 = t0 + h
            wr = tw_v[pl.ds(_TWC_OFF + _LOC_OFF[s] + r, 16)]
            wi = tw_v[pl.ds(_TWS_OFF + _LOC_OFF[s] + r, 16)]
            tr = re_v[pl.ds(t0, 16)]
            tii = im_v[pl.ds(t0, 16)]
            br = re_v[pl.ds(b0, 16)]
            bii = im_v[pl.ds(b0, 16)]
            xr = wr * br - wi * bii
            xi = wi * br + wr * bii
            re_v[pl.ds(t0, 16)] = tr + xr
            im_v[pl.ds(t0, 16)] = tii + xi
            re_v[pl.ds(b0, 16)] = tr - xr
            im_v[pl.ds(b0, 16)] = tii - xi
            return 0

        lax.fori_loop(0, 64, body, 0)

    base = pl.multiple_of(wid * CH, CH)
    pltpu.sync_copy(re_v, ore_hbm.at[pl.ds(base, CH)])
    pltpu.sync_copy(im_v, oim_hbm.at[pl.ds(base, CH)])


_k1 = functools.partial(
    pl.kernel,
    out_type=(jax.ShapeDtypeStruct((N,), jnp.float32),
              jax.ShapeDtypeStruct((N,), jnp.float32)),
    mesh=_MESH,
    compiler_params=pltpu.CompilerParams(needs_layout_passes=False),
    scratch_types=[
        pltpu.VMEM((LANES, 128), jnp.int32),
        pltpu.VMEM((CH,), jnp.float32),
        pltpu.VMEM((CH,), jnp.float32),
        pltpu.VMEM((_NCONST,), jnp.float32),
        pltpu.SemaphoreType.DMA,
    ],
)(_k1_body)


_NB = N // 2 // NCHUNK


def _stage_body(s, re_hbm, im_hbm, wr_hbm, wi_hbm, ore_hbm, oim_hbm,
                tre, tim, bre, bim, twr, twi):
    h = 1 << s
    wid = lax.axis_index("c") * 16 + lax.axis_index("s")
    b0 = wid * _NB
    t0 = pl.multiple_of(((b0 >> s) << (s + 1)) + (b0 & (h - 1)), _NB)
    j0 = pl.multiple_of(b0 & (h - 1), _NB)

    pltpu.sync_copy(re_hbm.at[pl.ds(t0, _NB)], tre)
    pltpu.sync_copy(im_hbm.at[pl.ds(t0, _NB)], tim)
    pltpu.sync_copy(re_hbm.at[pl.ds(t0 + h, _NB)], bre)
    pltpu.sync_copy(im_hbm.at[pl.ds(t0 + h, _NB)], bim)
    pltpu.sync_copy(wr_hbm.at[pl.ds(j0, _NB)], twr)
    pltpu.sync_copy(wi_hbm.at[pl.ds(j0, _NB)], twi)

    def body(k, _):
        o = k * 16
        tr = tre[pl.ds(o, 16)]
        tii = tim[pl.ds(o, 16)]
        br = bre[pl.ds(o, 16)]
        bii = bim[pl.ds(o, 16)]
        wr = twr[pl.ds(o, 16)]
        wi = twi[pl.ds(o, 16)]
        xr = wr * br - wi * bii
        xi = wi * br + wr * bii
        tre[pl.ds(o, 16)] = tr + xr
        tim[pl.ds(o, 16)] = tii + xi
        bre[pl.ds(o, 16)] = tr - xr
        bim[pl.ds(o, 16)] = tii - xi
        return 0

    lax.fori_loop(0, 64, body, 0)

    pltpu.sync_copy(tre, ore_hbm.at[pl.ds(t0, _NB)])
    pltpu.sync_copy(tim, oim_hbm.at[pl.ds(t0, _NB)])
    pltpu.sync_copy(bre, ore_hbm.at[pl.ds(t0 + h, _NB)])
    pltpu.sync_copy(bim, oim_hbm.at[pl.ds(t0 + h, _NB)])


def _make_stage(s):
    return functools.partial(
        pl.kernel,
        out_type=(jax.ShapeDtypeStruct((N,), jnp.float32),
                  jax.ShapeDtypeStruct((N,), jnp.float32)),
        mesh=_MESH,
        scratch_types=[pltpu.VMEM((_NB,), jnp.float32)] * 6,
    )(functools.partial(_stage_body, s))


_stage_kernels = {s: _make_stage(s) for s in range(11, 16)}


def kernel(x):
    re = x[:, 0]
    im = x[:, 1]
    idx = jnp.asarray(_BITREV_IDX)
    consts = jnp.asarray(_CONSTS)
    re, im = _k1(re, im, idx, consts)
    for s in range(11, 16):
        wr, wi = _CROSS[s]
        re, im = _stage_kernels[s](re, im, jnp.asarray(wr), jnp.asarray(wi))
    return jnp.stack((re, im), axis=-1)


# speedup vs baseline: 1.2690x; 1.2690x over previous
import functools
import math

import jax
import jax.numpy as jnp
import numpy as np
from jax import lax
from jax.experimental import pallas as pl
from jax.experimental.pallas import tpu as pltpu
from jax.experimental.pallas import tpu_sc as plsc

N = 65536
NCHUNK = 32
CH = 2048
LANES = 16


def _rev_bits(x, nbits):
    r = np.zeros_like(x)
    t = x.copy()
    for _ in range(nbits):
        r = (r << 1) | (t & 1)
        t >>= 1
    return r

_BITREV_IDX = _rev_bits(np.arange(N, dtype=np.int64), 16).reshape(
    NCHUNK, LANES, 128).astype(np.int32)

_lane = np.arange(LANES, dtype=np.int64)
_wr163, _wi163 = [], []
for _s in range(1, 4):
    _h = 1 << _s
    _ang = -2.0 * np.pi * (_lane & (_h - 1)) / (2 * _h)
    _wr163.append(np.cos(_ang))
    _wi163.append(np.sin(_ang))
_LOC_OFF = {}
_twc, _tws = [], []
_o = 0
for _s in range(4, 11):
    _h = 1 << _s
    _ang = -2.0 * np.pi * np.arange(_h, dtype=np.float64) / (2 * _h)
    _twc.append(np.cos(_ang))
    _tws.append(np.sin(_ang))
    _LOC_OFF[_s] = _o
    _o += _h
_NLOC = _o
_WR163_OFF = 0
_WI163_OFF = 48
_TWC_OFF = 96
_TWS_OFF = 96 + _NLOC
_NCONST = 96 + 2 * _NLOC
_CONSTS = np.concatenate(
    _wr163 + _wi163 + _twc + _tws).astype(np.float32)
assert _CONSTS.shape == (_NCONST,)

_CROSS = {}
for _s in range(11, 16):
    _h = 1 << _s
    _ang = -2.0 * np.pi * np.arange(_h, dtype=np.float64) / (2 * _h)
    _CROSS[_s] = (np.cos(_ang).astype(np.float32),
                  np.sin(_ang).astype(np.float32))

_XOFF = {}
_xwr, _xwi = [], []
_o = 0
for _q in range(4):
    _XOFF[_q] = _o
    _n = (1 << _q) * CH
    _ang = -2.0 * np.pi * np.arange(_n, dtype=np.float64) / (1 << (12 + _q))
    _xwr.append(np.cos(_ang))
    _xwi.append(np.sin(_ang))
    _o += _n
_NXTW = _o
_XWR = np.concatenate(_xwr).astype(np.float32)
_XWI = np.concatenate(_xwi).astype(np.float32)

_MESH = plsc.VectorSubcoreMesh(
    core_axis_name="c", subcore_axis_name="s", num_cores=2, num_subcores=16)


def _k1_body(re_hbm, im_hbm, idx_hbm, consts_hbm, xwr_hbm, xwi_hbm,
             ore_hbm, oim_hbm,
             idx_v, re_v, im_v, tw_v, pre_v, pim_v, xwr_v, xwi_v,
             shr_re, shr_im, sem):
    sid = lax.axis_index("s")
    wid = lax.axis_index("c") * 16 + sid

    pltpu.sync_copy(idx_hbm.at[wid], idx_v)
    pltpu.sync_copy(consts_hbm, tw_v)

    copies = []
    for j in range(LANES):
        copies.append(pltpu.make_async_copy(
            re_hbm.at[idx_v.at[j]], re_v.at[pl.ds(j * 128, 128)], sem))
        copies.append(pltpu.make_async_copy(
            im_hbm.at[idx_v.at[j]], im_v.at[pl.ds(j * 128, 128)], sem))
    for c in copies:
        c.start()
    for c in copies:
        c.wait()

    iota = lax.iota(jnp.int32, LANES)

    for s in range(0, 4):
        h = 1 << s
        pat = ((iota >> s) << (s + 1)) + (iota & (h - 1))
        if s > 0:
            wr = tw_v[pl.ds(_WR163_OFF + (s - 1) * 16, 16)]
            wi = tw_v[pl.ds(_WI163_OFF + (s - 1) * 16, 16)]

        def body03(k, _, s=s, h=h, pat=pat,
                   wr=(None if s == 0 else wr), wi=(None if s == 0 else wi)):
            ti = k * 32 + pat
            bi_ = ti + h
            tr = plsc.load_gather(re_v, [ti])
            tii = plsc.load_gather(im_v, [ti])
            br = plsc.load_gather(re_v, [bi_])
            bii = plsc.load_gather(im_v, [bi_])
            if s == 0:
                xr, xi = br, bii
            else:
                xr = wr * br - wi * bii
                xi = wi * br + wr * bii
            plsc.store_scatter(re_v, [ti], tr + xr)
            plsc.store_scatter(im_v, [ti], tii + xi)
            plsc.store_scatter(re_v, [bi_], tr - xr)
            plsc.store_scatter(im_v, [bi_], tii - xi)
            return 0

        lax.fori_loop(0, 64, body03, 0)

    for s in range(4, 11):
        h = 1 << s

        def body(k, _, s=s, h=h):
            b = k * 16
            r = b & (h - 1)
            t0 = ((b >> s) << (s + 1)) + r
            b0 = t0 + h
            wr = tw_v[pl.ds(_TWC_OFF + _LOC_OFF[s] + r, 16)]
            wi = tw_v[pl.ds(_TWS_OFF + _LOC_OFF[s] + r, 16)]
            tr = re_v[pl.ds(t0, 16)]
            tii = im_v[pl.ds(t0, 16)]
            br = re_v[pl.ds(b0, 16)]
            bii = im_v[pl.ds(b0, 16)]
            xr = wr * br - wi * bii
            xi = wi * br + wr * bii
            re_v[pl.ds(t0, 16)] = tr + xr
            im_v[pl.ds(t0, 16)] = tii + xi
            re_v[pl.ds(b0, 16)] = tr - xr
            im_v[pl.ds(b0, 16)] = tii - xi
            return 0

        lax.fori_loop(0, 64, body, 0)

    for q in range(4):
        off = pl.multiple_of((wid & ((1 << q) - 1)) * CH, CH)
        pltpu.sync_copy(xwr_hbm.at[pl.ds(_XOFF[q] + off, CH)],
                        xwr_v.at[pl.ds(q * CH, CH)])
        pltpu.sync_copy(xwi_hbm.at[pl.ds(_XOFF[q] + off, CH)],
                        xwi_v.at[pl.ds(q * CH, CH)])

    pltpu.sync_copy(re_v, shr_re.at[sid])
    pltpu.sync_copy(im_v, shr_im.at[sid])
    plsc.subcore_barrier()

    for q in range(4):
        psid = sid ^ (1 << q)
        b = q & 1
        pltpu.sync_copy(shr_re.at[b * 16 + psid], pre_v)
        pltpu.sync_copy(shr_im.at[b * 16 + psid], pim_v)
        mt = ((sid >> q) & 1 ^ 1).astype(jnp.float32)
        pt = 1.0 - mt
        sign = 2.0 * mt - 1.0

        def bodyx(k, _, q=q, mt=mt, pt=pt, sign=sign):
            o = k * 16
            mr = re_v[pl.ds(o, 16)]
            mi = im_v[pl.ds(o, 16)]
            pr = pre_v[pl.ds(o, 16)]
            pi = pim_v[pl.ds(o, 16)]
            wr = xwr_v[pl.ds(q * CH + o, 16)]
            wi = xwi_v[pl.ds(q * CH + o, 16)]
            tr = mt * mr + pt * pr
            tii = mt * mi + pt * pi
            br = mt * pr + pt * mr
            bii = mt * pi + pt * mi
            xr = wr * br - wi * bii
            xi = wi * br + wr * bii
            re_v[pl.ds(o, 16)] = tr + sign * xr
            im_v[pl.ds(o, 16)] = tii + sign * xi
            return 0

        lax.fori_loop(0, 128, bodyx, 0)
        if q < 3:
            nb = (q + 1) & 1
            pltpu.sync_copy(re_v, shr_re.at[nb * 16 + sid])
            pltpu.sync_copy(im_v, shr_im.at[nb * 16 + sid])
            plsc.subcore_barrier()

    base = pl.multiple_of(wid * CH, CH)
    pltpu.sync_copy(re_v, ore_hbm.at[pl.ds(base, CH)])
    pltpu.sync_copy(im_v, oim_hbm.at[pl.ds(base, CH)])


_k1 = functools.partial(
    pl.kernel,
    out_type=(jax.ShapeDtypeStruct((N,), jnp.float32),
              jax.ShapeDtypeStruct((N,), jnp.float32)),
    mesh=_MESH,
    compiler_params=pltpu.CompilerParams(needs_layout_passes=False),
    scratch_types=[
        pltpu.VMEM((LANES, 128), jnp.int32),
        pltpu.VMEM((CH,), jnp.float32),
        pltpu.VMEM((CH,), jnp.float32),
        pltpu.VMEM((_NCONST,), jnp.float32),
        pltpu.VMEM((CH,), jnp.float32),
        pltpu.VMEM((CH,), jnp.float32),
        pltpu.VMEM((4 * CH,), jnp.float32),
        pltpu.VMEM((4 * CH,), jnp.float32),
        pltpu.VMEM_SHARED((32, CH), jnp.float32),
        pltpu.VMEM_SHARED((32, CH), jnp.float32),
        pltpu.SemaphoreType.DMA,
    ],
)(_k1_body)


_NB = N // 2 // NCHUNK


def _stage_body(s, re_hbm, im_hbm, wr_hbm, wi_hbm, ore_hbm, oim_hbm,
                tre, tim, bre, bim, twr, twi):
    h = 1 << s
    wid = lax.axis_index("c") * 16 + lax.axis_index("s")
    b0 = wid * _NB
    t0 = pl.multiple_of(((b0 >> s) << (s + 1)) + (b0 & (h - 1)), _NB)
    j0 = pl.multiple_of(b0 & (h - 1), _NB)

    pltpu.sync_copy(re_hbm.at[pl.ds(t0, _NB)], tre)
    pltpu.sync_copy(im_hbm.at[pl.ds(t0, _NB)], tim)
    pltpu.sync_copy(re_hbm.at[pl.ds(t0 + h, _NB)], bre)
    pltpu.sync_copy(im_hbm.at[pl.ds(t0 + h, _NB)], bim)
    pltpu.sync_copy(wr_hbm.at[pl.ds(j0, _NB)], twr)
    pltpu.sync_copy(wi_hbm.at[pl.ds(j0, _NB)], twi)

    def body(k, _):
        o = k * 16
        tr = tre[pl.ds(o, 16)]
        tii = tim[pl.ds(o, 16)]
        br = bre[pl.ds(o, 16)]
        bii = bim[pl.ds(o, 16)]
        wr = twr[pl.ds(o, 16)]
        wi = twi[pl.ds(o, 16)]
        xr = wr * br - wi * bii
        xi = wi * br + wr * bii
        tre[pl.ds(o, 16)] = tr + xr
        tim[pl.ds(o, 16)] = tii + xi
        bre[pl.ds(o, 16)] = tr - xr
        bim[pl.ds(o, 16)] = tii - xi
        return 0

    lax.fori_loop(0, 64, body, 0)

    pltpu.sync_copy(tre, ore_hbm.at[pl.ds(t0, _NB)])
    pltpu.sync_copy(tim, oim_hbm.at[pl.ds(t0, _NB)])
    pltpu.sync_copy(bre, ore_hbm.at[pl.ds(t0 + h, _NB)])
    pltpu.sync_copy(bim, oim_hbm.at[pl.ds(t0 + h, _NB)])


def _make_stage(s):
    return functools.partial(
        pl.kernel,
        out_type=(jax.ShapeDtypeStruct((N,), jnp.float32),
                  jax.ShapeDtypeStruct((N,), jnp.float32)),
        mesh=_MESH,
        scratch_types=[pltpu.VMEM((_NB,), jnp.float32)] * 6,
    )(functools.partial(_stage_body, s))


_stage15 = _make_stage(15)


def kernel(x):
    re = x[:, 0]
    im = x[:, 1]
    idx = jnp.asarray(_BITREV_IDX)
    consts = jnp.asarray(_CONSTS)
    re, im = _k1(re, im, idx, consts, jnp.asarray(_XWR), jnp.asarray(_XWI))
    wr, wi = _CROSS[15]
    re, im = _stage15(re, im, jnp.asarray(wr), jnp.asarray(wi))
    return jnp.stack((re, im), axis=-1)
